# Initial kernel scaffold; baseline (speedup 1.0000x reference)
import jax
import jax.numpy as jnp
from jax.experimental import pallas as pl


def kernel(vision_tokens, text_tokens, attention_logits, attention_weights, Wp, bp, Wr, br, Ws, bs):
    raise NotImplementedError("write your pallas kernel here")


# speedup vs baseline: 3.1689x; 3.1689x over previous
import jax
import jax.numpy as jnp
from jax.experimental import pallas as pl
from jax.experimental.pallas import tpu as pltpu

_LV = 1024
_LT = 1024
_H = 16
_BR = 256
_NB = _LV // _BR
_LAMBDA_SAMPLE = 3.0
_TEMPERATURE = 10.0
_SAMPLING_RATE = 0.5


def _mask_kernel(al_ref, aw_ref, wp_ref, bp_ref, wr_ref, ws_ref, brbs_ref,
                 out_ref, acc_l, acc_w):
    h = pl.program_id(0)
    r = pl.program_id(1)

    @pl.when((h == 0) & (r == 0))
    def _init():
        acc_l[...] = jnp.zeros_like(acc_l)
        acc_w[...] = jnp.zeros_like(acc_w)

    acc_l[...] += al_ref[0]
    acc_w[...] += aw_ref[0]

    @pl.when((h == _H - 1) & (r == _NB - 1))
    def _epilogue():
        inv = 1.0 / (_H * _LV)
        col_self = jnp.sum(acc_l[...], axis=0, keepdims=True) * inv
        col_cross = jnp.sum(acc_w[...], axis=0, keepdims=True) * inv
        s_red = (col_self + col_cross) * 0.5

        s1 = jnp.sum(col_self) * (1.0 / _LV)
        s2 = jnp.sum(col_cross) * (1.0 / _LV)
        z = s1 * wp_ref[0:1, :] + s2 * wp_ref[1:2, :] + bp_ref[...]
        theta_r = jax.nn.sigmoid(jnp.sum(z * wr_ref[...]) + brbs_ref[0])
        theta_s = jax.nn.sigmoid(jnp.sum(z * ws_ref[...]) + brbs_ref[1])

        j = jax.lax.broadcasted_iota(jnp.int32, (1, _LV), 1)
        grid = 32
        stride = 2
        sampled = ((j // grid) % stride == 0) & ((j % grid) % stride == 0)
        s_sp = jnp.where(sampled, 1.0 - _SAMPLING_RATE * _LAMBDA_SAMPLE, 0.0)

        mask_r = jax.nn.sigmoid((s_red - theta_r) * _TEMPERATURE)
        mask_s = jax.nn.sigmoid((s_sp - theta_s) * _TEMPERATURE)
        out_ref[...] = jnp.maximum(mask_r, mask_s)


def kernel(vision_tokens, text_tokens, attention_logits, attention_weights,
           Wp, bp, Wr, br, Ws, bs):
    B, Lv, D = vision_tokens.shape
    al = attention_logits.reshape(_H, 2 * _LV, 2 * _LV)
    aw = attention_weights.reshape(_H, 2 * _LV, 2 * _LV)
    wp_t = Wp.T
    bp2 = bp.reshape(1, D)
    brbs = jnp.concatenate([br, bs])

    out = pl.pallas_call(
        _mask_kernel,
        grid=(_H, _NB),
        in_specs=[
            pl.BlockSpec((1, _BR, _LV), lambda h, r: (h, r, 0)),
            pl.BlockSpec((1, _BR, _LV), lambda h, r: (h, r + _NB, 0)),
            pl.BlockSpec((2, D), lambda h, r: (0, 0)),
            pl.BlockSpec((1, D), lambda h, r: (0, 0)),
            pl.BlockSpec((1, D), lambda h, r: (0, 0)),
            pl.BlockSpec((1, D), lambda h, r: (0, 0)),
            pl.BlockSpec(memory_space=pltpu.SMEM),
        ],
        out_specs=pl.BlockSpec((1, _LV), lambda h, r: (0, 0)),
        out_shape=jax.ShapeDtypeStruct((1, _LV), jnp.float32),
        scratch_shapes=[
            pltpu.VMEM((_BR, _LV), jnp.float32),
            pltpu.VMEM((_BR, _LV), jnp.float32),
        ],
    )(al, aw, wp_t, bp2, Wr, Ws, brbs)
    return out
